# Initial kernel scaffold; baseline (speedup 1.0000x reference)
import jax
import jax.numpy as jnp
from jax.experimental import pallas as pl


def kernel(in_repr, group_idx, deltas):
    raise NotImplementedError("write your pallas kernel here")


# speedup vs baseline: 1.6730x; 1.6730x over previous
import functools

import jax
import jax.numpy as jnp
from jax import lax
from jax.experimental import pallas as pl
from jax.experimental.pallas import tpu as pltpu
from jax.experimental.pallas import tpu_sc as plsc

B = 16384
D = 128
NW = 32
CH = 128
ROWS_PER_W = B // NW
NCH = ROWS_PER_W // CH
LANES = 16


def _sc_body(in_hbm, idx_hbm, deltas_hbm, out_hbm, idx_v, gbuf, xbuf, sem):
    c = lax.axis_index("c")
    s = lax.axis_index("s")
    wid = s * 2 + c
    base = wid * ROWS_PER_W

    pltpu.sync_copy(idx_hbm.at[pl.ds(wid * NCH, NCH)], idx_v)

    for j in range(NCH):
        row0 = base + j * CH
        pltpu.async_copy(deltas_hbm.at[idx_v.at[j]], gbuf, sem).wait()
        pltpu.sync_copy(in_hbm.at[pl.ds(row0, CH)], xbuf)

        def body(i, _):
            for k in range(D // LANES):
                sl = pl.ds(k * LANES, LANES)
                xbuf[i, sl] = xbuf[i, sl] + gbuf[i, sl]
            return 0

        lax.fori_loop(0, CH, body, 0)
        pltpu.sync_copy(xbuf, out_hbm.at[pl.ds(row0, CH)])


@jax.jit
def kernel(in_repr, group_idx, deltas):
    idx2 = group_idx.astype(jnp.int32).reshape(B // CH, CH)
    mesh = plsc.VectorSubcoreMesh(core_axis_name="c", subcore_axis_name="s")
    fn = functools.partial(
        pl.kernel,
        mesh=mesh,
        out_type=jax.ShapeDtypeStruct((B, D), jnp.float32),
        scratch_types=[
            pltpu.VMEM((NCH, CH), jnp.int32),
            pltpu.VMEM((CH, D), jnp.float32),
            pltpu.VMEM((CH, D), jnp.float32),
            pltpu.SemaphoreType.DMA,
        ],
    )(_sc_body)
    return fn(in_repr, idx2, deltas)


# speedup vs baseline: 1.8996x; 1.1355x over previous
import functools

import jax
import jax.numpy as jnp
from jax import lax
from jax.experimental import pallas as pl
from jax.experimental.pallas import tpu as pltpu
from jax.experimental.pallas import tpu_sc as plsc

B = 16384
D = 128
NW = 32
CH = 128
ROWS_PER_W = B // NW
NCH = ROWS_PER_W // CH
LANES = 16


def _sc_body(in_hbm, idx_hbm, deltas_hbm, out_hbm, idx_v, gbuf, xbuf, sem):
    c = lax.axis_index("c")
    s = lax.axis_index("s")
    wid = s * 2 + c
    base = wid * ROWS_PER_W

    pltpu.sync_copy(idx_hbm.at[pl.ds(wid * NCH, NCH)], idx_v)

    for j in range(NCH):
        row0 = base + j * CH
        pltpu.sync_copy(in_hbm.at[pl.ds(row0, CH)], xbuf)
        pltpu.async_copy(deltas_hbm.at[idx_v.at[j]], xbuf, sem, add=True).wait()
        pltpu.sync_copy(xbuf, out_hbm.at[pl.ds(row0, CH)])


@jax.jit
def kernel(in_repr, group_idx, deltas):
    idx2 = group_idx.astype(jnp.int32).reshape(B // CH, CH)
    mesh = plsc.VectorSubcoreMesh(core_axis_name="c", subcore_axis_name="s")
    fn = functools.partial(
        pl.kernel,
        mesh=mesh,
        out_type=jax.ShapeDtypeStruct((B, D), jnp.float32),
        scratch_types=[
            pltpu.VMEM((NCH, CH), jnp.int32),
            pltpu.VMEM((CH, D), jnp.float32),
            pltpu.VMEM((CH, D), jnp.float32),
            pltpu.SemaphoreType.DMA,
        ],
    )(_sc_body)
    return fn(in_repr, idx2, deltas)


# speedup vs baseline: 2.0752x; 1.0924x over previous
import functools

import jax
import jax.numpy as jnp
from jax import lax
from jax.experimental import pallas as pl
from jax.experimental.pallas import tpu as pltpu
from jax.experimental.pallas import tpu_sc as plsc

B = 16384
D = 128
NW = 32
CH = 128
ROWS_PER_W = B // NW
NCH = ROWS_PER_W // CH
LANES = 16


def _sc_body(in_hbm, idx_hbm, deltas_hbm, out_hbm, idx_v,
             b0, b1, b2, b3,
             xs0, xs1, xs2, xs3, gs0, gs1, gs2, gs3, os0, os1, os2, os3):
    bufs = [b0, b1, b2, b3]
    xsems = [xs0, xs1, xs2, xs3]
    gsems = [gs0, gs1, gs2, gs3]
    osems = [os0, os1, os2, os3]
    c = lax.axis_index("c")
    s = lax.axis_index("s")
    wid = s * 2 + c
    base = wid * ROWS_PER_W

    xd = [pltpu.async_copy(in_hbm.at[pl.ds(base + j * CH, CH)], bufs[j], xsems[j])
          for j in range(NCH)]
    pltpu.sync_copy(idx_hbm.at[pl.ds(wid * NCH, NCH)], idx_v)
    gd = []
    for j in range(NCH):
        xd[j].wait()
        gd.append(pltpu.async_copy(deltas_hbm.at[idx_v.at[j]], bufs[j],
                                   gsems[j], add=True))
    od = []
    for j in range(NCH):
        gd[j].wait()
        od.append(pltpu.async_copy(bufs[j], out_hbm.at[pl.ds(base + j * CH, CH)],
                                   osems[j]))
    for j in range(NCH):
        od[j].wait()


@jax.jit
def kernel(in_repr, group_idx, deltas):
    idx2 = group_idx.astype(jnp.int32).reshape(B // CH, CH)
    mesh = plsc.VectorSubcoreMesh(core_axis_name="c", subcore_axis_name="s")
    fn = functools.partial(
        pl.kernel,
        mesh=mesh,
        out_type=jax.ShapeDtypeStruct((B, D), jnp.float32),
        scratch_types=(
            [pltpu.VMEM((NCH, CH), jnp.int32)]
            + [pltpu.VMEM((CH, D), jnp.float32) for _ in range(NCH)]
            + [pltpu.SemaphoreType.DMA for _ in range(3 * NCH)]
        ),
    )(_sc_body)
    return fn(in_repr, idx2, deltas)


# speedup vs baseline: 2.3492x; 1.1321x over previous
import functools

import jax
import jax.numpy as jnp
from jax import lax
from jax.experimental import pallas as pl
from jax.experimental.pallas import tpu as pltpu
from jax.experimental.pallas import tpu_sc as plsc

B = 16384
D = 128
G = 1000
NW = 32
CH = 128
ROWS_PER_W = B // NW
NCH = ROWS_PER_W // CH


def _sc_body(in_hbm, idx_hbm, deltas_hbm, out_hbm, idx_v, tbl_s,
             b0, b1, b2, b3,
             xs0, xs1, xs2, xs3, gs0, gs1, gs2, gs3, os0, os1, os2, os3):
    bufs = [b0, b1, b2, b3]
    xsems = [xs0, xs1, xs2, xs3]
    gsems = [gs0, gs1, gs2, gs3]
    osems = [os0, os1, os2, os3]
    c = lax.axis_index("c")
    s = lax.axis_index("s")
    wid = s * 2 + c
    base = wid * ROWS_PER_W

    xd = [pltpu.async_copy(in_hbm.at[pl.ds(base + j * CH, CH)], bufs[j], xsems[j])
          for j in range(NCH)]
    pltpu.sync_copy(idx_hbm.at[pl.ds(wid * NCH, NCH)], idx_v)

    @pl.when(s == 0)
    def _load_table():
        pltpu.sync_copy(deltas_hbm, tbl_s)

    plsc.subcore_barrier()
    gd = []
    for j in range(NCH):
        xd[j].wait()
        gd.append(pltpu.async_copy(tbl_s.at[idx_v.at[j]], bufs[j],
                                   gsems[j], add=True))
    od = []
    for j in range(NCH):
        gd[j].wait()
        od.append(pltpu.async_copy(bufs[j], out_hbm.at[pl.ds(base + j * CH, CH)],
                                   osems[j]))
    for j in range(NCH):
        od[j].wait()


@jax.jit
def kernel(in_repr, group_idx, deltas):
    idx2 = group_idx.astype(jnp.int32).reshape(B // CH, CH)
    mesh = plsc.VectorSubcoreMesh(core_axis_name="c", subcore_axis_name="s")
    fn = functools.partial(
        pl.kernel,
        mesh=mesh,
        out_type=jax.ShapeDtypeStruct((B, D), jnp.float32),
        scratch_types=(
            [pltpu.VMEM((NCH, CH), jnp.int32),
             pltpu.VMEM_SHARED((G, D), jnp.float32)]
            + [pltpu.VMEM((CH, D), jnp.float32) for _ in range(NCH)]
            + [pltpu.SemaphoreType.DMA for _ in range(3 * NCH)]
        ),
    )(_sc_body)
    return fn(in_repr, idx2, deltas)
